# Initial kernel scaffold; baseline (speedup 1.0000x reference)
import jax
import jax.numpy as jnp
from jax.experimental import pallas as pl


def kernel(mol_attr, sub_input, x_nodes, edge_index, edge_attr, batch_ids, W_atom, Wb, W1, b1, W2, b2, Wv1, bv1, Wv2, bv2, Wm_in, bm_in, Wm_h, bm_h, Wl_in, bl_in, Wl_h, bl_h, Wl_out, bl_out):
    raise NotImplementedError("write your pallas kernel here")


# speedup vs baseline: 5.4317x; 5.4317x over previous
import functools

import numpy as np
import jax
import jax.numpy as jnp
from jax import lax
from jax.experimental import pallas as pl
from jax.experimental.pallas import tpu as pltpu

EMB = 300
EMBP = 384
HALF = 192
HID = 600
HIDP = 640
NN = 10000
NB = 64
NE = 160000
RB = 1000
NRB = NN // RB
BN_S = float(1.0 / np.sqrt(1.0 + 1e-5))


def _pad2(a, r, c):
    out = jnp.zeros((r, c), a.dtype)
    return out.at[: a.shape[0], : a.shape[1]].set(a)


def _pad1(a, n):
    out = jnp.zeros((n,), a.dtype)
    return out.at[: a.shape[0]].set(a)


def _enc_body(x_ref, wa_ref, out_ref):
    x = x_ref[0]
    acc = jnp.zeros((RB, EMBP), jnp.float32)
    for c in range(9):
        oh = (x[:, c][:, None] == lax.broadcasted_iota(jnp.int32, (1, 128), 1)
              ).astype(jnp.float32)
        acc = acc + jnp.dot(oh, wa_ref[c], preferred_element_type=jnp.float32)
    out_ref[0] = acc[:, :HALF]
    out_ref[1] = acc[:, HALF:]


def _encode(x3, wap):
    return pl.pallas_call(
        _enc_body,
        grid=(NRB,),
        in_specs=[
            pl.BlockSpec((1, RB, 9), lambda i: (i, 0, 0)),
            pl.BlockSpec((9, 128, EMBP), lambda i: (0, 0, 0)),
        ],
        out_specs=pl.BlockSpec((2, RB, HALF), lambda i: (0, i, 0)),
        out_shape=jax.ShapeDtypeStruct((2, NN, HALF), jnp.float32),
    )(x3, wap)


def _ebuild_body(wb_ref, out_ref):
    i = lax.broadcasted_iota(jnp.int32, (4096, 1), 0)
    acc = jnp.zeros((4096, HALF), jnp.float32)
    for j, dig in enumerate((i // 256, (i // 16) % 16, i % 16)):
        oh = (dig == lax.broadcasted_iota(jnp.int32, (1, 16), 1)
              ).astype(jnp.float32)
        acc = acc + jnp.dot(oh, wb_ref[0, 0, j], preferred_element_type=jnp.float32)
    out_ref[0, 0] = acc


def _ebuild(wbsp):
    return pl.pallas_call(
        _ebuild_body,
        grid=(3, 2),
        in_specs=[pl.BlockSpec((1, 1, 3, 16, HALF), lambda l, c: (l, c, 0, 0, 0))],
        out_specs=pl.BlockSpec((1, 1, 4096, HALF), lambda l, c: (l, c, 0, 0)),
        out_shape=jax.ShapeDtypeStruct((3, 2, 4096, HALF), jnp.float32),
    )(wbsp)


def _vn_body(h_ref, bid_ref, vn_ref, w1_ref, b1_ref, w2_ref, b2_ref, out_ref):
    i = pl.program_id(0)
    x = jnp.concatenate([h_ref[0], h_ref[1]], axis=1)
    bid = bid_ref[0, 0]
    p = (bid[:, None] == lax.broadcasted_iota(jnp.int32, (1, NB), 1)
         ).astype(jnp.float32)
    contrib = lax.dot_general(p, x, (((0,), (0,)), ((), ())),
                              preferred_element_type=jnp.float32)

    @pl.when(i == 0)
    def _():
        out_ref[...] = contrib

    @pl.when(i > 0)
    def _():
        out_ref[...] = out_ref[...] + contrib

    @pl.when(i == NRB - 1)
    def _():
        vt = out_ref[...] + vn_ref[...]
        vt = jnp.maximum((jnp.dot(vt, w1_ref[...], preferred_element_type=jnp.float32)
                          + b1_ref[...]) * BN_S, 0.0)
        out_ref[...] = jnp.maximum(
            (jnp.dot(vt, w2_ref[...], preferred_element_type=jnp.float32)
             + b2_ref[...]) * BN_S, 0.0)


def _vn_update(hs, bid3, vn, wv1, bv1, wv2, bv2):
    return pl.pallas_call(
        _vn_body,
        grid=(NRB,),
        in_specs=[
            pl.BlockSpec((2, RB, HALF), lambda i: (0, i, 0)),
            pl.BlockSpec((1, 1, RB), lambda i: (i, 0, 0)),
            pl.BlockSpec((NB, EMBP), lambda i: (0, 0)),
            pl.BlockSpec((EMBP, HIDP), lambda i: (0, 0)),
            pl.BlockSpec((1, HIDP), lambda i: (0, 0)),
            pl.BlockSpec((HIDP, EMBP), lambda i: (0, 0)),
            pl.BlockSpec((1, EMBP), lambda i: (0, 0)),
        ],
        out_specs=pl.BlockSpec((NB, EMBP), lambda i: (0, 0)),
        out_shape=jax.ShapeDtypeStruct((NB, EMBP), jnp.float32),
    )(hs, bid3, vn, wv1, bv1, wv2, bv2)


def _mlp_body(h_ref, agg_ref, bid_ref, vn_ref, w1_ref, b1_ref, w2_ref, b2_ref,
              out_ref):
    x = jnp.concatenate([h_ref[0] + agg_ref[0], h_ref[1] + agg_ref[1]], axis=1)
    z = jnp.maximum((jnp.dot(x, w1_ref[...], preferred_element_type=jnp.float32)
                     + b1_ref[...]) * BN_S, 0.0)
    z = (jnp.dot(z, w2_ref[...], preferred_element_type=jnp.float32)
         + b2_ref[...]) * BN_S
    z = jnp.maximum(z, 0.0)
    bid = bid_ref[0, 0]
    p = (bid[:, None] == lax.broadcasted_iota(jnp.int32, (1, NB), 1)
         ).astype(jnp.float32)
    z = z + jnp.dot(p, vn_ref[...], preferred_element_type=jnp.float32)
    out_ref[0] = z[:, :HALF]
    out_ref[1] = z[:, HALF:]


def _mlp_layer(hs, agg, bid3, vn_new, w1, b1, w2, b2):
    return pl.pallas_call(
        _mlp_body,
        grid=(NRB,),
        in_specs=[
            pl.BlockSpec((2, RB, HALF), lambda i: (0, i, 0)),
            pl.BlockSpec((2, RB, HALF), lambda i: (0, i, 0)),
            pl.BlockSpec((1, 1, RB), lambda i: (i, 0, 0)),
            pl.BlockSpec((NB, EMBP), lambda i: (0, 0)),
            pl.BlockSpec((EMBP, HIDP), lambda i: (0, 0)),
            pl.BlockSpec((1, HIDP), lambda i: (0, 0)),
            pl.BlockSpec((HIDP, EMBP), lambda i: (0, 0)),
            pl.BlockSpec((1, EMBP), lambda i: (0, 0)),
        ],
        out_specs=pl.BlockSpec((2, RB, HALF), lambda i: (0, i, 0)),
        out_shape=jax.ShapeDtypeStruct((2, NN, HALF), jnp.float32),
    )(hs, agg, bid3, vn_new, w1, b1, w2, b2)


def _mlp_last_body(h_ref, agg_ref, bid_ref, w1_ref, b1_ref, w2_ref, b2_ref,
                   out_ref):
    i = pl.program_id(0)
    x = jnp.concatenate([h_ref[0] + agg_ref[0], h_ref[1] + agg_ref[1]], axis=1)
    z = jnp.maximum((jnp.dot(x, w1_ref[...], preferred_element_type=jnp.float32)
                     + b1_ref[...]) * BN_S, 0.0)
    z = (jnp.dot(z, w2_ref[...], preferred_element_type=jnp.float32)
         + b2_ref[...]) * BN_S
    bid = bid_ref[0, 0]
    p = (bid[:, None] == lax.broadcasted_iota(jnp.int32, (1, NB), 1)
         ).astype(jnp.float32)
    contrib = lax.dot_general(p, z, (((0,), (0,)), ((), ())),
                              preferred_element_type=jnp.float32)

    @pl.when(i == 0)
    def _():
        out_ref[...] = contrib

    @pl.when(i > 0)
    def _():
        out_ref[...] = out_ref[...] + contrib


def _mlp_last(hs, agg, bid3, w1, b1, w2, b2):
    return pl.pallas_call(
        _mlp_last_body,
        grid=(NRB,),
        in_specs=[
            pl.BlockSpec((2, RB, HALF), lambda i: (0, i, 0)),
            pl.BlockSpec((2, RB, HALF), lambda i: (0, i, 0)),
            pl.BlockSpec((1, 1, RB), lambda i: (i, 0, 0)),
            pl.BlockSpec((EMBP, HIDP), lambda i: (0, 0)),
            pl.BlockSpec((1, HIDP), lambda i: (0, 0)),
            pl.BlockSpec((HIDP, EMBP), lambda i: (0, 0)),
            pl.BlockSpec((1, EMBP), lambda i: (0, 0)),
        ],
        out_specs=pl.BlockSpec((NB, EMBP), lambda i: (0, 0)),
        out_shape=jax.ShapeDtypeStruct((NB, EMBP), jnp.float32),
    )(hs, agg, bid3, w1, b1, w2, b2)


def _head_body(t0_ref, hg_ref, wmi_ref, bmi_ref, wmh_ref, bmh_ref,
               wla_ref, wlb_ref, bli_ref, wlh_ref, blh_ref, wlo_ref, blo_ref,
               out_ref):
    t = jnp.maximum((jnp.dot(t0_ref[...], wmi_ref[...],
                             preferred_element_type=jnp.float32)
                     + bmi_ref[...]) * BN_S, 0.0)
    for i in range(2):
        t = jnp.maximum((jnp.dot(t, wmh_ref[i], preferred_element_type=jnp.float32)
                         + bmh_ref[i]) * BN_S, 0.0)
    x2 = jnp.minimum(t, 50.0)
    u = (jnp.dot(hg_ref[...], wla_ref[...], preferred_element_type=jnp.float32)
         + jnp.dot(x2, wlb_ref[...], preferred_element_type=jnp.float32)
         + bli_ref[...]) * BN_S
    u = jnp.maximum(u, 0.0)
    for i in range(2):
        u = jnp.maximum((jnp.dot(u, wlh_ref[i], preferred_element_type=jnp.float32)
                         + blh_ref[i]) * BN_S, 0.0)
    out_ref[...] = (jnp.dot(u, wlo_ref[...], preferred_element_type=jnp.float32)
                    + blo_ref[...])


def _head(t0p, hg, wmi, bmi, wmh, bmh, wla, wlb, bli, wlh, blh, wlo, blo):
    full = lambda *s: pl.BlockSpec(s, lambda: tuple(0 for _ in s))
    return pl.pallas_call(
        _head_body,
        in_specs=[
            full(NB, 64), full(NB, EMBP),
            full(64, EMBP), full(1, EMBP), full(2, EMBP, EMBP), full(2, 1, EMBP),
            full(EMBP, EMBP), full(EMBP, EMBP), full(1, EMBP),
            full(2, EMBP, EMBP), full(2, 1, EMBP), full(EMBP, 128), full(1, 128),
        ],
        out_specs=full(NB, 128),
        out_shape=jax.ShapeDtypeStruct((NB, 128), jnp.float32),
    )(t0p, hg, wmi, bmi, wmh, bmh, wla, wlb, bli, wlh, blh, wlo, blo)


def _edge_agg(hs, e_half, src, dst, combo):
    h = jnp.concatenate([hs[0], hs[1]], axis=1)
    e = jnp.concatenate([e_half[0], e_half[1]], axis=1)
    msg = jnp.maximum(h[src] + e[combo], 0.0)
    agg = jax.ops.segment_sum(msg, dst, num_segments=NN)
    return jnp.stack([agg[:, :HALF], agg[:, HALF:]])


def kernel(mol_attr, sub_input, x_nodes, edge_index, edge_attr, batch_ids,
           W_atom, Wb, W1, b1, W2, b2, Wv1, bv1, Wv2, bv2,
           Wm_in, bm_in, Wm_h, bm_h, Wl_in, bl_in, Wl_h, bl_h, Wl_out, bl_out):
    f32 = jnp.float32
    wap = jnp.zeros((9, 128, EMBP), f32).at[:, :, :EMB].set(W_atom)
    wbsp = jnp.zeros((3, 3, 16, EMBP), f32).at[:, :, :, :EMB].set(Wb)
    wbsp = wbsp.reshape(3, 3, 16, 2, HALF).transpose(0, 3, 1, 2, 4)
    w1p = jnp.zeros((3, EMBP, HIDP), f32).at[:, :EMB, :HID].set(W1)
    b1p = jnp.zeros((3, 1, HIDP), f32).at[:, 0, :HID].set(b1)
    w2p = jnp.zeros((3, HIDP, EMBP), f32).at[:, :HID, :EMB].set(W2)
    b2p = jnp.zeros((3, 1, EMBP), f32).at[:, 0, :EMB].set(b2)
    wv1p = jnp.zeros((2, EMBP, HIDP), f32).at[:, :EMB, :HID].set(Wv1)
    bv1p = jnp.zeros((2, 1, HIDP), f32).at[:, 0, :HID].set(bv1)
    wv2p = jnp.zeros((2, HIDP, EMBP), f32).at[:, :HID, :EMB].set(Wv2)
    bv2p = jnp.zeros((2, 1, EMBP), f32).at[:, 0, :EMB].set(bv2)
    t0 = jnp.concatenate([sub_input, mol_attr], axis=1)
    t0p = _pad2(t0, NB, 64)
    wmi = _pad2(Wm_in, 64, EMBP)
    bmi = _pad1(bm_in, EMBP)[None, :]
    wmh = jnp.zeros((2, EMBP, EMBP), f32).at[:, :EMB, :EMB].set(Wm_h)
    bmh = jnp.zeros((2, 1, EMBP), f32).at[:, 0, :EMB].set(bm_h)
    wla = _pad2(Wl_in[:EMB], EMBP, EMBP)
    wlb = _pad2(Wl_in[EMB:], EMBP, EMBP)
    bli = _pad1(bl_in, EMBP)[None, :]
    wlh = jnp.zeros((2, EMBP, EMBP), f32).at[:, :EMB, :EMB].set(Wl_h)
    blh = jnp.zeros((2, 1, EMBP), f32).at[:, 0, :EMB].set(bl_h)
    wlo = _pad2(Wl_out, EMBP, 128)
    blo = _pad1(bl_out, 128)[None, :]

    x3 = x_nodes.astype(jnp.int32).reshape(NRB, RB, 9)
    bid3 = batch_ids.astype(jnp.int32).reshape(NRB, 1, RB)
    src = edge_index[0].astype(jnp.int32)
    dst = edge_index[1].astype(jnp.int32)
    ea = edge_attr.astype(jnp.int32)
    combo = ea[:, 0] * 256 + ea[:, 1] * 16 + ea[:, 2]

    e_split = _ebuild(wbsp)
    hs = _encode(x3, wap)
    vn = jnp.zeros((NB, EMBP), f32)
    for l in range(3):
        agg = _edge_agg(hs, e_split[l], src, dst, combo)
        if l < 2:
            vn = _vn_update(hs, bid3, vn, wv1p[l], bv1p[l], wv2p[l], bv2p[l])
            hs = _mlp_layer(hs, agg, bid3, vn, w1p[l], b1p[l], w2p[l], b2p[l])
        else:
            hg = _mlp_last(hs, agg, bid3, w1p[l], b1p[l], w2p[l], b2p[l])
    out = _head(t0p, hg, wmi, bmi, wmh, bmh, wla, wlb, bli, wlh, blh, wlo, blo)
    return out[:, :1]


# speedup vs baseline: 14.4338x; 2.6573x over previous
import functools

import numpy as np
import jax
import jax.numpy as jnp
from jax import lax
from jax.experimental import pallas as pl
from jax.experimental.pallas import tpu as pltpu
from jax.experimental.pallas import tpu_sc as plsc

EMB = 300
EMBP = 384
HALF = 192
QCOL = 64
NQ = 6
HID = 600
HIDP = 640
NN = 10000
NB = 64
NE = 160000
RB = 1000
NRB = NN // RB
BN_S = float(1.0 / np.sqrt(1.0 + 1e-5))


def _pad2(a, r, c):
    out = jnp.zeros((r, c), a.dtype)
    return out.at[: a.shape[0], : a.shape[1]].set(a)


def _pad1(a, n):
    out = jnp.zeros((n,), a.dtype)
    return out.at[: a.shape[0]].set(a)


def _enc_body(x_ref, wa_ref, out_ref):
    x = x_ref[0]
    acc = jnp.zeros((RB, EMBP), jnp.float32)
    for c in range(9):
        oh = (x[:, c][:, None] == lax.broadcasted_iota(jnp.int32, (1, 128), 1)
              ).astype(jnp.float32)
        acc = acc + jnp.dot(oh, wa_ref[c], preferred_element_type=jnp.float32)
    for q in range(NQ):
        out_ref[q] = acc[:, q * QCOL:(q + 1) * QCOL]


def _encode(x3, wap):
    return pl.pallas_call(
        _enc_body,
        grid=(NRB,),
        in_specs=[
            pl.BlockSpec((1, RB, 9), lambda i: (i, 0, 0)),
            pl.BlockSpec((9, 128, EMBP), lambda i: (0, 0, 0)),
        ],
        out_specs=pl.BlockSpec((NQ, RB, QCOL), lambda i: (0, i, 0)),
        out_shape=jax.ShapeDtypeStruct((NQ, NN, QCOL), jnp.float32),
    )(x3, wap)


def _ebuild_body(wb_ref, out_ref):
    i = lax.broadcasted_iota(jnp.int32, (4096, 1), 0)
    acc = jnp.zeros((4096, QCOL), jnp.float32)
    for j, dig in enumerate((i // 256, (i // 16) % 16, i % 16)):
        oh = (dig == lax.broadcasted_iota(jnp.int32, (1, 16), 1)
              ).astype(jnp.float32)
        acc = acc + jnp.dot(oh, wb_ref[0, 0, j], preferred_element_type=jnp.float32)
    out_ref[0, 0] = acc


def _ebuild(wbsp):
    return pl.pallas_call(
        _ebuild_body,
        grid=(3, NQ),
        in_specs=[pl.BlockSpec((1, 1, 3, 16, QCOL), lambda l, c: (l, c, 0, 0, 0))],
        out_specs=pl.BlockSpec((1, 1, 4096, QCOL), lambda l, c: (l, c, 0, 0)),
        out_shape=jax.ShapeDtypeStruct((3, NQ, 4096, QCOL), jnp.float32),
    )(wbsp)


def _vn_body(h_ref, bid_ref, vn_ref, w1_ref, b1_ref, w2_ref, b2_ref, out_ref):
    i = pl.program_id(0)
    x = jnp.concatenate([h_ref[q] for q in range(NQ)], axis=1)
    bid = bid_ref[0, 0]
    p = (bid[:, None] == lax.broadcasted_iota(jnp.int32, (1, NB), 1)
         ).astype(jnp.float32)
    contrib = lax.dot_general(p, x, (((0,), (0,)), ((), ())),
                              preferred_element_type=jnp.float32)

    @pl.when(i == 0)
    def _():
        out_ref[...] = contrib

    @pl.when(i > 0)
    def _():
        out_ref[...] = out_ref[...] + contrib

    @pl.when(i == NRB - 1)
    def _():
        vt = out_ref[...] + vn_ref[...]
        vt = jnp.maximum((jnp.dot(vt, w1_ref[...], preferred_element_type=jnp.float32)
                          + b1_ref[...]) * BN_S, 0.0)
        out_ref[...] = jnp.maximum(
            (jnp.dot(vt, w2_ref[...], preferred_element_type=jnp.float32)
             + b2_ref[...]) * BN_S, 0.0)


def _vn_update(hs, bid3, vn, wv1, bv1, wv2, bv2):
    return pl.pallas_call(
        _vn_body,
        grid=(NRB,),
        in_specs=[
            pl.BlockSpec((NQ, RB, QCOL), lambda i: (0, i, 0)),
            pl.BlockSpec((1, 1, RB), lambda i: (i, 0, 0)),
            pl.BlockSpec((NB, EMBP), lambda i: (0, 0)),
            pl.BlockSpec((EMBP, HIDP), lambda i: (0, 0)),
            pl.BlockSpec((1, HIDP), lambda i: (0, 0)),
            pl.BlockSpec((HIDP, EMBP), lambda i: (0, 0)),
            pl.BlockSpec((1, EMBP), lambda i: (0, 0)),
        ],
        out_specs=pl.BlockSpec((NB, EMBP), lambda i: (0, 0)),
        out_shape=jax.ShapeDtypeStruct((NB, EMBP), jnp.float32),
    )(hs, bid3, vn, wv1, bv1, wv2, bv2)


def _mlp_body(h_ref, agg_ref, bid_ref, vn_ref, w1_ref, b1_ref, w2_ref, b2_ref,
              out_ref):
    x = jnp.concatenate([h_ref[q] + agg_ref[q] for q in range(NQ)], axis=1)
    z = jnp.maximum((jnp.dot(x, w1_ref[...], preferred_element_type=jnp.float32)
                     + b1_ref[...]) * BN_S, 0.0)
    z = (jnp.dot(z, w2_ref[...], preferred_element_type=jnp.float32)
         + b2_ref[...]) * BN_S
    z = jnp.maximum(z, 0.0)
    bid = bid_ref[0, 0]
    p = (bid[:, None] == lax.broadcasted_iota(jnp.int32, (1, NB), 1)
         ).astype(jnp.float32)
    z = z + jnp.dot(p, vn_ref[...], preferred_element_type=jnp.float32)
    for q in range(NQ):
        out_ref[q] = z[:, q * QCOL:(q + 1) * QCOL]


def _mlp_layer(hs, agg, bid3, vn_new, w1, b1, w2, b2):
    return pl.pallas_call(
        _mlp_body,
        grid=(NRB,),
        in_specs=[
            pl.BlockSpec((NQ, RB, QCOL), lambda i: (0, i, 0)),
            pl.BlockSpec((NQ, RB, QCOL), lambda i: (0, i, 0)),
            pl.BlockSpec((1, 1, RB), lambda i: (i, 0, 0)),
            pl.BlockSpec((NB, EMBP), lambda i: (0, 0)),
            pl.BlockSpec((EMBP, HIDP), lambda i: (0, 0)),
            pl.BlockSpec((1, HIDP), lambda i: (0, 0)),
            pl.BlockSpec((HIDP, EMBP), lambda i: (0, 0)),
            pl.BlockSpec((1, EMBP), lambda i: (0, 0)),
        ],
        out_specs=pl.BlockSpec((NQ, RB, QCOL), lambda i: (0, i, 0)),
        out_shape=jax.ShapeDtypeStruct((NQ, NN, QCOL), jnp.float32),
    )(hs, agg, bid3, vn_new, w1, b1, w2, b2)


def _mlp_last_body(h_ref, agg_ref, bid_ref, w1_ref, b1_ref, w2_ref, b2_ref,
                   out_ref):
    i = pl.program_id(0)
    x = jnp.concatenate([h_ref[q] + agg_ref[q] for q in range(NQ)], axis=1)
    z = jnp.maximum((jnp.dot(x, w1_ref[...], preferred_element_type=jnp.float32)
                     + b1_ref[...]) * BN_S, 0.0)
    z = (jnp.dot(z, w2_ref[...], preferred_element_type=jnp.float32)
         + b2_ref[...]) * BN_S
    bid = bid_ref[0, 0]
    p = (bid[:, None] == lax.broadcasted_iota(jnp.int32, (1, NB), 1)
         ).astype(jnp.float32)
    contrib = lax.dot_general(p, z, (((0,), (0,)), ((), ())),
                              preferred_element_type=jnp.float32)

    @pl.when(i == 0)
    def _():
        out_ref[...] = contrib

    @pl.when(i > 0)
    def _():
        out_ref[...] = out_ref[...] + contrib


def _mlp_last(hs, agg, bid3, w1, b1, w2, b2):
    return pl.pallas_call(
        _mlp_last_body,
        grid=(NRB,),
        in_specs=[
            pl.BlockSpec((NQ, RB, QCOL), lambda i: (0, i, 0)),
            pl.BlockSpec((NQ, RB, QCOL), lambda i: (0, i, 0)),
            pl.BlockSpec((1, 1, RB), lambda i: (i, 0, 0)),
            pl.BlockSpec((EMBP, HIDP), lambda i: (0, 0)),
            pl.BlockSpec((1, HIDP), lambda i: (0, 0)),
            pl.BlockSpec((HIDP, EMBP), lambda i: (0, 0)),
            pl.BlockSpec((1, EMBP), lambda i: (0, 0)),
        ],
        out_specs=pl.BlockSpec((NB, EMBP), lambda i: (0, 0)),
        out_shape=jax.ShapeDtypeStruct((NB, EMBP), jnp.float32),
    )(hs, agg, bid3, w1, b1, w2, b2)


def _head_body(t0_ref, hg_ref, wmi_ref, bmi_ref, wmh_ref, bmh_ref,
               wla_ref, wlb_ref, bli_ref, wlh_ref, blh_ref, wlo_ref, blo_ref,
               out_ref):
    t = jnp.maximum((jnp.dot(t0_ref[...], wmi_ref[...],
                             preferred_element_type=jnp.float32)
                     + bmi_ref[...]) * BN_S, 0.0)
    for i in range(2):
        t = jnp.maximum((jnp.dot(t, wmh_ref[i], preferred_element_type=jnp.float32)
                         + bmh_ref[i]) * BN_S, 0.0)
    x2 = jnp.minimum(t, 50.0)
    u = (jnp.dot(hg_ref[...], wla_ref[...], preferred_element_type=jnp.float32)
         + jnp.dot(x2, wlb_ref[...], preferred_element_type=jnp.float32)
         + bli_ref[...]) * BN_S
    u = jnp.maximum(u, 0.0)
    for i in range(2):
        u = jnp.maximum((jnp.dot(u, wlh_ref[i], preferred_element_type=jnp.float32)
                         + blh_ref[i]) * BN_S, 0.0)
    out_ref[...] = (jnp.dot(u, wlo_ref[...], preferred_element_type=jnp.float32)
                    + blo_ref[...])


def _head(t0p, hg, wmi, bmi, wmh, bmh, wla, wlb, bli, wlh, blh, wlo, blo):
    full = lambda *s: pl.BlockSpec(s, lambda: tuple(0 for _ in s))
    return pl.pallas_call(
        _head_body,
        in_specs=[
            full(NB, 64), full(NB, EMBP),
            full(64, EMBP), full(1, EMBP), full(2, EMBP, EMBP), full(2, 1, EMBP),
            full(EMBP, EMBP), full(EMBP, EMBP), full(1, EMBP),
            full(2, EMBP, EMBP), full(2, 1, EMBP), full(EMBP, 128), full(1, 128),
        ],
        out_specs=full(NB, 128),
        out_shape=jax.ShapeDtypeStruct((NB, 128), jnp.float32),
    )(t0p, hg, wmi, bmi, wmh, bmh, wla, wlb, bli, wlh, blh, wlo, blo)


NTIL = 16
EPT = NE // NTIL
CH = 125
NCHK = EPT // CH
NV = QCOL // 16
ZC = 128
NZC = NN // ZC
ZTAIL = NN - NZC * ZC


def _sc_edge_body(h_hbm, e_hbm, srcA, comboA, dstA, out_hbm,
                  srcb, combob, dstb, hbuf, ebuf, msgb, zbuf, aggsh,
                  sem1, sem2):
    cid = lax.axis_index("c")
    sid = lax.axis_index("s")
    pltpu.sync_copy(dstA.at[sid], dstb)

    def zloop(i, _):
        for v in range(NV):
            zbuf[i, pl.ds(v * 16, 16)] = jnp.zeros((16,), jnp.float32)
        return 0
    lax.fori_loop(0, ZC, zloop, 0)

    for p in range(3):
        q = cid * 3 + p
        pltpu.sync_copy(srcA.at[q, sid], srcb)
        pltpu.sync_copy(comboA.at[q, sid], combob)
        for k in range(5):
            c = sid + k * NTIL

            @pl.when(c < NZC)
            def _():
                pltpu.sync_copy(zbuf, aggsh.at[pl.ds(c * ZC, ZC)])

            @pl.when(c == NZC)
            def _():
                pltpu.sync_copy(zbuf.at[pl.ds(0, ZTAIL)],
                                aggsh.at[pl.ds(NZC * ZC, ZTAIL)])
        plsc.subcore_barrier()

        def chunk(j, _):
            cp1 = pltpu.async_copy(h_hbm.at[srcb.at[j]], hbuf, sem1)
            cp2 = pltpu.async_copy(e_hbm.at[combob.at[j]], ebuf, sem2)
            cp1.wait()
            cp2.wait()

            def edge(i, _):
                for v in range(NV):
                    sl = pl.ds(v * 16, 16)
                    msgb[i, sl] = jnp.maximum(hbuf[i, sl] + ebuf[i, sl], 0.0)
                return 0
            lax.fori_loop(0, CH, edge, 0)
            pltpu.sync_copy(msgb, aggsh.at[dstb.at[j]], add=True)
            return 0
        lax.fori_loop(0, NCHK, chunk, 0)

        plsc.subcore_barrier()
        for k in range(5):
            c = sid + k * NTIL

            @pl.when(c < NZC)
            def _():
                pltpu.sync_copy(aggsh.at[pl.ds(c * ZC, ZC)],
                                out_hbm.at[q, pl.ds(c * ZC, ZC)])

            @pl.when(c == NZC)
            def _():
                pltpu.sync_copy(aggsh.at[pl.ds(NZC * ZC, ZTAIL)],
                                out_hbm.at[q, pl.ds(NZC * ZC, ZTAIL)])
        plsc.subcore_barrier()


_sc_edge = pl.kernel(
    _sc_edge_body,
    out_type=jax.ShapeDtypeStruct((NQ, NN, QCOL), jnp.float32),
    mesh=plsc.VectorSubcoreMesh(core_axis_name="c", subcore_axis_name="s"),
    compiler_params=pltpu.CompilerParams(use_tc_tiling_on_sc=False),
    scratch_types=[
        pltpu.VMEM((NCHK, CH), jnp.int32),
        pltpu.VMEM((NCHK, CH), jnp.int32),
        pltpu.VMEM((NCHK, CH), jnp.int32),
        pltpu.VMEM((CH, QCOL), jnp.float32),
        pltpu.VMEM((CH, QCOL), jnp.float32),
        pltpu.VMEM((CH, QCOL), jnp.float32),
        pltpu.VMEM((ZC, QCOL), jnp.float32),
        pltpu.VMEM_SHARED((NN, QCOL), jnp.float32),
        pltpu.SemaphoreType.DMA,
        pltpu.SemaphoreType.DMA,
    ],
)


def _edge_agg(h_flat, e_flat, src_aug, combo_aug, dst_t):
    return _sc_edge(h_flat, e_flat, src_aug, combo_aug, dst_t)


def kernel(mol_attr, sub_input, x_nodes, edge_index, edge_attr, batch_ids,
           W_atom, Wb, W1, b1, W2, b2, Wv1, bv1, Wv2, bv2,
           Wm_in, bm_in, Wm_h, bm_h, Wl_in, bl_in, Wl_h, bl_h, Wl_out, bl_out):
    f32 = jnp.float32
    wap = jnp.zeros((9, 128, EMBP), f32).at[:, :, :EMB].set(W_atom)
    wbsp = jnp.zeros((3, 3, 16, EMBP), f32).at[:, :, :, :EMB].set(Wb)
    wbsp = wbsp.reshape(3, 3, 16, NQ, QCOL).transpose(0, 3, 1, 2, 4)
    w1p = jnp.zeros((3, EMBP, HIDP), f32).at[:, :EMB, :HID].set(W1)
    b1p = jnp.zeros((3, 1, HIDP), f32).at[:, 0, :HID].set(b1)
    w2p = jnp.zeros((3, HIDP, EMBP), f32).at[:, :HID, :EMB].set(W2)
    b2p = jnp.zeros((3, 1, EMBP), f32).at[:, 0, :EMB].set(b2)
    wv1p = jnp.zeros((2, EMBP, HIDP), f32).at[:, :EMB, :HID].set(Wv1)
    bv1p = jnp.zeros((2, 1, HIDP), f32).at[:, 0, :HID].set(bv1)
    wv2p = jnp.zeros((2, HIDP, EMBP), f32).at[:, :HID, :EMB].set(Wv2)
    bv2p = jnp.zeros((2, 1, EMBP), f32).at[:, 0, :EMB].set(bv2)
    t0 = jnp.concatenate([sub_input, mol_attr], axis=1)
    t0p = _pad2(t0, NB, 64)
    wmi = _pad2(Wm_in, 64, EMBP)
    bmi = _pad1(bm_in, EMBP)[None, :]
    wmh = jnp.zeros((2, EMBP, EMBP), f32).at[:, :EMB, :EMB].set(Wm_h)
    bmh = jnp.zeros((2, 1, EMBP), f32).at[:, 0, :EMB].set(bm_h)
    wla = _pad2(Wl_in[:EMB], EMBP, EMBP)
    wlb = _pad2(Wl_in[EMB:], EMBP, EMBP)
    bli = _pad1(bl_in, EMBP)[None, :]
    wlh = jnp.zeros((2, EMBP, EMBP), f32).at[:, :EMB, :EMB].set(Wl_h)
    blh = jnp.zeros((2, 1, EMBP), f32).at[:, 0, :EMB].set(bl_h)
    wlo = _pad2(Wl_out, EMBP, 128)
    blo = _pad1(bl_out, 128)[None, :]

    x3 = x_nodes.astype(jnp.int32).reshape(NRB, RB, 9)
    bid3 = batch_ids.astype(jnp.int32).reshape(NRB, 1, RB)
    src = edge_index[0].astype(jnp.int32)
    dst = edge_index[1].astype(jnp.int32)
    ea = edge_attr.astype(jnp.int32)
    combo = ea[:, 0] * 256 + ea[:, 1] * 16 + ea[:, 2]
    src_t = src.reshape(NTIL, NCHK, CH)
    src_aug = jnp.stack([src_t + q * NN for q in range(NQ)])
    combo_t = combo.reshape(NTIL, NCHK, CH)
    combo_aug = jnp.stack([combo_t + q * 4096 for q in range(NQ)])
    dst_t = dst.reshape(NTIL, NCHK, CH)

    e_split = _ebuild(wbsp)
    hs = _encode(x3, wap)
    vn = jnp.zeros((NB, EMBP), f32)
    for l in range(3):
        agg = _edge_agg(hs.reshape(NQ * NN, QCOL),
                        e_split[l].reshape(NQ * 4096, QCOL),
                        src_aug, combo_aug, dst_t)
        if l < 2:
            vn = _vn_update(hs, bid3, vn, wv1p[l], bv1p[l], wv2p[l], bv2p[l])
            hs = _mlp_layer(hs, agg, bid3, vn, w1p[l], b1p[l], w2p[l], b2p[l])
        else:
            hg = _mlp_last(hs, agg, bid3, w1p[l], b1p[l], w2p[l], b2p[l])
    out = _head(t0p, hg, wmi, bmi, wmh, bmh, wla, wlb, bli, wlh, blh, wlo, blo)
    return out[:, :1]
